# baseline (speedup 1.0000x reference)
import functools

import jax
import jax.numpy as jnp
from jax import lax
from jax.experimental import pallas as pl
from jax.experimental.pallas import tpu as pltpu
from jax.experimental.pallas import tpu_sc as plsc

D = 1024
SEQ = 4096
BATCH = 4
NW = 32
SW = SEQ // NW
T = 32
NT = SW // T
LANES = 16

_mesh = plsc.VectorSubcoreMesh(core_axis_name="c", subcore_axis_name="s")


@functools.partial(
    pl.kernel,
    mesh=_mesh,
    out_type=jax.ShapeDtypeStruct((BATCH * SEQ, D), jnp.float32),
    scratch_types=[
        pltpu.VMEM((T, D), jnp.float32),
        pltpu.VMEM((T, D), jnp.float32),
        pltpu.VMEM((T,), jnp.int32),
        pltpu.VMEM((LANES,), jnp.int32),
        pltpu.SemaphoreType.DMA,
        pltpu.SemaphoreType.DMA,
    ],
)
def _sc_pe_add(x_hbm, pe_hbm, sp_hbm, out_hbm, pebuf, xbuf, idxbuf, spbuf,
               gsem, xsem):
    wid = lax.axis_index("s") * 2 + lax.axis_index("c")
    s0 = wid * SW
    pltpu.sync_copy(sp_hbm, spbuf)
    vsp = spbuf[...]

    for t in range(NT):
        base = s0 + t * T
        for k in range(T // LANES):
            idxbuf[pl.ds(k * LANES, LANES)] = (
                vsp + lax.iota(jnp.int32, LANES) + (base + k * LANES)
            )
        pltpu.async_copy(pe_hbm.at[idxbuf], pebuf, gsem).wait()
        for b in range(BATCH):
            row0 = b * SEQ + base
            pltpu.async_copy(x_hbm.at[pl.ds(row0, T)], xbuf, xsem).wait()

            def add_row(r, _):
                def add_vec(k2, _):
                    sl = pl.ds(k2 * LANES, LANES)
                    xbuf[r, sl] = xbuf[r, sl] + pebuf[r, sl]
                    return 0
                return lax.fori_loop(0, D // LANES, add_vec, 0)

            lax.fori_loop(0, T, add_row, 0)
            pltpu.async_copy(xbuf, out_hbm.at[pl.ds(row0, T)], xsem).wait()


@jax.jit
def _pe_add(x, pos_embedding, sp16):
    batch, seq, d = x.shape
    xf = x.reshape(batch * seq, d)
    out = _sc_pe_add(xf, pos_embedding, sp16)
    return out.reshape(x.shape)


def kernel(x, pos_embedding, start_pos):
    sp16 = jnp.full((LANES,), start_pos, dtype=jnp.int32)
    return _pe_add(x, pos_embedding, sp16)


# speedup vs baseline: 3.2781x; 3.2781x over previous
import functools

import jax
import jax.numpy as jnp
from jax import lax
from jax.experimental import pallas as pl
from jax.experimental.pallas import tpu as pltpu
from jax.experimental.pallas import tpu_sc as plsc

D = 1024
SEQ = 4096
BATCH = 4
LANES = 16

SC_SEQ = 512
NW = 32
SW = SC_SEQ // NW
T = 16
NT = SW // T

_mesh = plsc.VectorSubcoreMesh(core_axis_name="c", subcore_axis_name="s")


@functools.partial(
    pl.kernel,
    mesh=_mesh,
    out_type=jax.ShapeDtypeStruct((BATCH * SC_SEQ, D), jnp.float32),
    scratch_types=[
        pltpu.VMEM((T, D), jnp.float32),
        pltpu.VMEM((T, D), jnp.float32),
        pltpu.VMEM((T,), jnp.int32),
        pltpu.VMEM((LANES,), jnp.int32),
        pltpu.SemaphoreType.DMA,
        pltpu.SemaphoreType.DMA,
    ],
)
def _sc_pe_add(x_hbm, pe_hbm, sp_hbm, out_hbm, pebuf, xbuf, idxbuf, spbuf,
               gsem, xsem):
    wid = lax.axis_index("s") * 2 + lax.axis_index("c")
    s0 = wid * SW
    pltpu.sync_copy(sp_hbm, spbuf)
    vsp = spbuf[...]

    for t in range(NT):
        base = s0 + t * T
        for k in range(T // LANES):
            idxbuf[pl.ds(k * LANES, LANES)] = (
                vsp + lax.iota(jnp.int32, LANES) + (base + k * LANES)
            )
        pltpu.async_copy(pe_hbm.at[idxbuf], pebuf, gsem).wait()
        for b in range(BATCH):
            row0 = b * SEQ + base
            pltpu.async_copy(x_hbm.at[pl.ds(row0, T)], xbuf, xsem).wait()

            def add_row(r, _):
                def add_vec(k2, _):
                    sl = pl.ds(k2 * LANES, LANES)
                    xbuf[r, sl] = xbuf[r, sl] + pebuf[r, sl]
                    return 0
                return lax.fori_loop(0, D // LANES, add_vec, 0)

            lax.fori_loop(0, T, add_row, 0)
            pltpu.async_copy(
                xbuf, out_hbm.at[pl.ds(b * SC_SEQ + base, T)], xsem
            ).wait()


R = 512
K = 8
TC_SEQ = SEQ - SC_SEQ
NJ = TC_SEQ // R
NC = BATCH * NJ


def _tc_body(sp_ref, x_any, pe_any, o_any, xbuf, pebuf, obuf, sx, spe, so):
    def rows(c):
        b, j = divmod(c, NJ)
        return b * SEQ + SC_SEQ + j * R

    def x_copy(c):
        return pltpu.make_async_copy(
            x_any.at[pl.ds(rows(c), R)], xbuf.at[c % K], sx.at[c % K]
        )

    def pe_copy(j):
        start = pl.multiple_of(sp_ref[0] + SC_SEQ + j * R, 8)
        return pltpu.make_async_copy(
            pe_any.at[pl.ds(start, R)], pebuf.at[j], spe.at[j]
        )

    def o_copy(c):
        return pltpu.make_async_copy(
            obuf.at[c % K], o_any.at[pl.ds(rows(c), R)], so.at[c % K]
        )

    x_copy(0).start()
    pe_copy(0).start()
    for i in range(1, K):
        x_copy(i).start()
    for j in range(1, NJ):
        pe_copy(j).start()

    for c in range(NC):
        if c >= K:
            o_copy(c - K).wait()
        x_copy(c).wait()
        if c < NJ:
            pe_copy(c).wait()
        obuf[c % K] = xbuf[c % K] + pebuf[c % NJ]
        o_copy(c).start()
        if c + K < NC:
            x_copy(c + K).start()

    for c in range(NC - K, NC):
        o_copy(c).wait()


def _tc_call(sp, xf, pos_embedding):
    grid_spec = pltpu.PrefetchScalarGridSpec(
        num_scalar_prefetch=1,
        in_specs=[
            pl.BlockSpec(memory_space=pl.ANY),
            pl.BlockSpec(memory_space=pl.ANY),
        ],
        out_specs=pl.BlockSpec(memory_space=pl.ANY),
        scratch_shapes=[
            pltpu.VMEM((K, R, D), jnp.float32),
            pltpu.VMEM((NJ, R, D), jnp.float32),
            pltpu.VMEM((K, R, D), jnp.float32),
            pltpu.SemaphoreType.DMA((K,)),
            pltpu.SemaphoreType.DMA((NJ,)),
            pltpu.SemaphoreType.DMA((K,)),
        ],
    )
    return pl.pallas_call(
        _tc_body,
        grid_spec=grid_spec,
        out_shape=jax.ShapeDtypeStruct(xf.shape, xf.dtype),
        compiler_params=pltpu.CompilerParams(
            vmem_limit_bytes=60 * 1024 * 1024,
        ),
    )(sp, xf, pos_embedding)


@jax.jit
def _pe_add(x, pos_embedding, sp, sp16):
    batch, seq, d = x.shape
    xf = x.reshape(batch * seq, d)
    out_tc = _tc_call(sp, xf, pos_embedding)
    out_sc = _sc_pe_add(xf, pos_embedding, sp16)
    return lax.dynamic_update_slice(
        out_tc.reshape(x.shape),
        out_sc.reshape(batch, SC_SEQ, d),
        (0, 0, 0),
    )


def kernel(x, pos_embedding, start_pos):
    sp = jnp.atleast_1d(jnp.asarray(start_pos, dtype=jnp.int32))
    sp16 = jnp.full((LANES,), start_pos, dtype=jnp.int32)
    return _pe_add(x, pos_embedding, sp, sp16)


# speedup vs baseline: 3.2886x; 1.0032x over previous
import functools

import jax
import jax.numpy as jnp
from jax import lax
from jax.experimental import pallas as pl
from jax.experimental.pallas import tpu as pltpu
from jax.experimental.pallas import tpu_sc as plsc

D = 1024
SEQ = 4096
BATCH = 4
LANES = 16

SC_SEQ = 512
NW = 32
SW = SC_SEQ // NW
T = 16
NT = SW // T

_mesh = plsc.VectorSubcoreMesh(core_axis_name="c", subcore_axis_name="s")


@functools.partial(
    pl.kernel,
    mesh=_mesh,
    out_type=jax.ShapeDtypeStruct((BATCH * SC_SEQ, D), jnp.float32),
    scratch_types=[
        pltpu.VMEM((T, D), jnp.float32),
        pltpu.VMEM((T, D), jnp.float32),
        pltpu.VMEM((T,), jnp.int32),
        pltpu.VMEM((LANES,), jnp.int32),
        pltpu.SemaphoreType.DMA,
        pltpu.SemaphoreType.DMA,
    ],
)
def _sc_pe_add(x_hbm, pe_hbm, sp_hbm, out_hbm, pebuf, xbuf, idxbuf, spbuf,
               gsem, xsem):
    wid = lax.axis_index("s") * 2 + lax.axis_index("c")
    s0 = wid * SW
    pltpu.sync_copy(sp_hbm, spbuf)
    vsp = spbuf[...]

    for t in range(NT):
        base = s0 + t * T
        for k in range(T // LANES):
            idxbuf[pl.ds(k * LANES, LANES)] = (
                vsp + lax.iota(jnp.int32, LANES) + (base + k * LANES)
            )
        pltpu.async_copy(pe_hbm.at[idxbuf], pebuf, gsem).wait()
        for b in range(BATCH):
            row0 = b * SEQ + base
            pltpu.async_copy(x_hbm.at[pl.ds(row0, T)], xbuf, xsem).wait()

            def add_row(r, _):
                def add_vec(k2, _):
                    sl = pl.ds(k2 * LANES, LANES)
                    xbuf[r, sl] = xbuf[r, sl] + pebuf[r, sl]
                    return 0
                return lax.fori_loop(0, D // LANES, add_vec, 0)

            lax.fori_loop(0, T, add_row, 0)
            pltpu.async_copy(
                xbuf, out_hbm.at[pl.ds(b * SC_SEQ + base, T)], xsem
            ).wait()


R = 512
K = 8
TC_SEQ = SEQ - SC_SEQ
NJ = TC_SEQ // R
NC = BATCH * NJ


def _tc_body(sp_ref, x_any, pe_any, o_any, xbuf, pebuf, obuf, sx, spe, so):
    def rows(c):
        b, j = divmod(c, NJ)
        return b * SEQ + SC_SEQ + j * R

    def x_copy(c):
        return pltpu.make_async_copy(
            x_any.at[pl.ds(rows(c), R)], xbuf.at[c % K], sx.at[c % K]
        )

    def pe_copy(j):
        start = pl.multiple_of(sp_ref[0] + SC_SEQ + j * R, 8)
        return pltpu.make_async_copy(
            pe_any.at[pl.ds(start, R)], pebuf.at[j], spe.at[j]
        )

    def o_copy(c):
        return pltpu.make_async_copy(
            obuf.at[c % K], o_any.at[pl.ds(rows(c), R)], so.at[c % K]
        )

    x_copy(0).start()
    pe_copy(0).start()
    for i in range(1, K):
        x_copy(i).start()
    for j in range(1, NJ):
        pe_copy(j).start()

    for c in range(NC):
        if c >= K:
            o_copy(c - K).wait()
        x_copy(c).wait()
        if c < NJ:
            pe_copy(c).wait()
        obuf[c % K] = xbuf[c % K] + pebuf[c % NJ]
        o_copy(c).start()
        if c + K < NC:
            x_copy(c + K).start()

    for c in range(NC - K, NC):
        o_copy(c).wait()


def _tc_call(sp, xf, pos_embedding):
    grid_spec = pltpu.PrefetchScalarGridSpec(
        num_scalar_prefetch=1,
        in_specs=[
            pl.BlockSpec(memory_space=pl.ANY),
            pl.BlockSpec(memory_space=pl.ANY),
        ],
        out_specs=pl.BlockSpec(memory_space=pl.ANY),
        scratch_shapes=[
            pltpu.VMEM((K, R, D), jnp.float32),
            pltpu.VMEM((NJ, R, D), jnp.float32),
            pltpu.VMEM((K, R, D), jnp.float32),
            pltpu.SemaphoreType.DMA((K,)),
            pltpu.SemaphoreType.DMA((NJ,)),
            pltpu.SemaphoreType.DMA((K,)),
        ],
    )
    return pl.pallas_call(
        _tc_body,
        grid_spec=grid_spec,
        out_shape=jax.ShapeDtypeStruct(xf.shape, xf.dtype),
        compiler_params=pltpu.CompilerParams(
            vmem_limit_bytes=60 * 1024 * 1024,
        ),
    )(sp, xf, pos_embedding)


@jax.jit
def _pe_add(x, pos_embedding, sp, sp16):
    batch, seq, d = x.shape
    xf = x.reshape(batch * seq, d)
    out_sc = _sc_pe_add(xf, pos_embedding, sp16)
    out_tc = _tc_call(sp, xf, pos_embedding)
    return lax.dynamic_update_slice(
        out_tc.reshape(x.shape),
        out_sc.reshape(batch, SC_SEQ, d),
        (0, 0, 0),
    )


def kernel(x, pos_embedding, start_pos):
    sp = jnp.atleast_1d(jnp.asarray(start_pos, dtype=jnp.int32))
    sp16 = jnp.full((LANES,), start_pos, dtype=jnp.int32)
    return _pe_add(x, pos_embedding, sp, sp16)


# speedup vs baseline: 5.1015x; 1.5513x over previous
import jax
import jax.numpy as jnp
from jax.experimental import pallas as pl
from jax.experimental.pallas import tpu as pltpu

D = 1024
R = 1024
K = 5
NPE = 4096 // R


def _body(sp_ref, x_any, pe_any, o_any, xbuf, pebuf, obuf, sx, spe, so):
    n = 16384 // R

    def x_copy(c):
        return pltpu.make_async_copy(
            x_any.at[pl.ds(c * R, R)], xbuf.at[c % K], sx.at[c % K]
        )

    def pe_copy(q):
        start = pl.multiple_of(sp_ref[0] + q * R, 8)
        return pltpu.make_async_copy(
            pe_any.at[pl.ds(start, R)], pebuf.at[q], spe.at[q]
        )

    def o_copy(c):
        return pltpu.make_async_copy(
            obuf.at[c % K], o_any.at[pl.ds(c * R, R)], so.at[c % K]
        )

    x_copy(0).start()
    pe_copy(0).start()
    for i in range(1, K):
        x_copy(i).start()
    for q in range(1, NPE):
        pe_copy(q).start()

    for c in range(n):
        if c >= K:
            o_copy(c - K).wait()
        x_copy(c).wait()
        if c < NPE:
            pe_copy(c).wait()
        obuf[c % K] = xbuf[c % K] + pebuf[c % NPE]
        o_copy(c).start()
        if c + K < n:
            x_copy(c + K).start()

    for c in range(n - K, n):
        o_copy(c).wait()


@jax.jit
def _pe_add(sp, x, pos_embedding):
    batch, seq, d = x.shape
    xf = x.reshape(batch * seq, d)
    grid_spec = pltpu.PrefetchScalarGridSpec(
        num_scalar_prefetch=1,
        in_specs=[
            pl.BlockSpec(memory_space=pl.ANY),
            pl.BlockSpec(memory_space=pl.ANY),
        ],
        out_specs=pl.BlockSpec(memory_space=pl.ANY),
        scratch_shapes=[
            pltpu.VMEM((K, R, d), jnp.float32),
            pltpu.VMEM((NPE, R, d), jnp.float32),
            pltpu.VMEM((K, R, d), jnp.float32),
            pltpu.SemaphoreType.DMA((K,)),
            pltpu.SemaphoreType.DMA((NPE,)),
            pltpu.SemaphoreType.DMA((K,)),
        ],
    )
    out = pl.pallas_call(
        _body,
        grid_spec=grid_spec,
        out_shape=jax.ShapeDtypeStruct(xf.shape, x.dtype),
        compiler_params=pltpu.CompilerParams(
            vmem_limit_bytes=60 * 1024 * 1024,
        ),
    )(sp, xf, pos_embedding)
    return out.reshape(x.shape)


def kernel(x, pos_embedding, start_pos):
    sp = jnp.atleast_1d(jnp.asarray(start_pos, dtype=jnp.int32))
    return _pe_add(sp, x, pos_embedding)


# speedup vs baseline: 5.1575x; 1.0110x over previous
import jax
import jax.numpy as jnp
from jax.experimental import pallas as pl
from jax.experimental.pallas import tpu as pltpu

D = 1024
R = 2048
K = 3
KO = 2
NPE = 4096 // R


def _body(sp_ref, x_any, pe_any, o_any, xbuf, pebuf, obuf, sx, spe, so):
    n = 16384 // R

    def x_copy(c):
        return pltpu.make_async_copy(
            x_any.at[pl.ds(c * R, R)], xbuf.at[c % K], sx.at[c % K]
        )

    def pe_copy(q):
        start = pl.multiple_of(sp_ref[0] + q * R, 8)
        return pltpu.make_async_copy(
            pe_any.at[pl.ds(start, R)], pebuf.at[q], spe.at[q]
        )

    def o_copy(c):
        return pltpu.make_async_copy(
            obuf.at[c % KO], o_any.at[pl.ds(c * R, R)], so.at[c % KO]
        )

    x_copy(0).start()
    pe_copy(0).start()
    for i in range(1, K):
        x_copy(i).start()
    for q in range(1, NPE):
        pe_copy(q).start()

    for c in range(n):
        if c >= KO:
            o_copy(c - KO).wait()
        x_copy(c).wait()
        if c < NPE:
            pe_copy(c).wait()
        obuf[c % KO] = xbuf[c % K] + pebuf[c % NPE]
        o_copy(c).start()
        if c + K < n:
            x_copy(c + K).start()

    for c in range(n - KO, n):
        o_copy(c).wait()


@jax.jit
def _pe_add(sp, x, pos_embedding):
    batch, seq, d = x.shape
    xf = x.reshape(batch * seq, d)
    grid_spec = pltpu.PrefetchScalarGridSpec(
        num_scalar_prefetch=1,
        in_specs=[
            pl.BlockSpec(memory_space=pl.ANY),
            pl.BlockSpec(memory_space=pl.ANY),
        ],
        out_specs=pl.BlockSpec(memory_space=pl.ANY),
        scratch_shapes=[
            pltpu.VMEM((K, R, d), jnp.float32),
            pltpu.VMEM((NPE, R, d), jnp.float32),
            pltpu.VMEM((KO, R, d), jnp.float32),
            pltpu.SemaphoreType.DMA((K,)),
            pltpu.SemaphoreType.DMA((NPE,)),
            pltpu.SemaphoreType.DMA((KO,)),
        ],
    )
    out = pl.pallas_call(
        _body,
        grid_spec=grid_spec,
        out_shape=jax.ShapeDtypeStruct(xf.shape, x.dtype),
        compiler_params=pltpu.CompilerParams(
            vmem_limit_bytes=62 * 1024 * 1024,
        ),
    )(sp, xf, pos_embedding)
    return out.reshape(x.shape)


def kernel(x, pos_embedding, start_pos):
    sp = jnp.atleast_1d(jnp.asarray(start_pos, dtype=jnp.int32))
    return _pe_add(sp, x, pos_embedding)
